# baseline (speedup 1.0000x reference)
import jax
import jax.numpy as jnp
from jax.experimental import pallas as pl
from jax.experimental.pallas import tpu as pltpu
from jax.experimental.pallas import tpu_sc as plsc

F = 128
NCORES = 2
NSUB = 16
CPW = 2 * F // (NCORES * NSUB)
LANES = 16


def kernel(mask, row_embed, col_embed):
    b, h, w = mask.shape
    mesh = plsc.VectorSubcoreMesh(core_axis_name="c", subcore_axis_name="s")

    @pl.kernel(
        out_type=jax.ShapeDtypeStruct((b, 2 * F, h, w), jnp.float32),
        mesh=mesh,
        compiler_params=pltpu.CompilerParams(
            use_tc_tiling_on_sc=False, needs_layout_passes=False
        ),
        scratch_types=[
            pltpu.VMEM((CPW, h, w), jnp.float32),
            pltpu.VMEM((h, F), jnp.float32),
            pltpu.SemaphoreType.DMA,
        ],
    )
    def sc_kernel(row_hbm, col_hbm, out_hbm, chunk, tab, sem):
        core = jax.lax.axis_index("c")
        sub = jax.lax.axis_index("s")
        c0 = core * F + sub * CPW
        iota = jax.lax.iota(jnp.int32, LANES)

        @pl.when(core == 0)
        def _x_half():
            pltpu.async_copy(col_hbm.at[pl.ds(0, w)], tab, sem).wait()
            for j in range(CPW):
                cvec = jnp.zeros((LANES,), jnp.int32) + (c0 + j)
                for xh in range(w // LANES):
                    xs = iota + xh * LANES
                    v = plsc.load_gather(tab, [xs, cvec])
                    for y in range(h):
                        chunk.at[j].at[y][pl.ds(xh * LANES, LANES)] = v

        @pl.when(core == 1)
        def _y_half():
            pltpu.async_copy(row_hbm.at[pl.ds(0, h)], tab, sem).wait()
            for j in range(CPW):
                ccvec = jnp.zeros((LANES,), jnp.int32) + (c0 + j - F)
                for y in range(h):
                    yvec = jnp.zeros((LANES,), jnp.int32) + y
                    v = plsc.load_gather(tab, [yvec, ccvec])
                    for xh in range(w // LANES):
                        chunk.at[j].at[y][pl.ds(xh * LANES, LANES)] = v

        copies = [
            pltpu.make_async_copy(chunk, out_hbm.at[bi, pl.ds(c0, CPW)], sem)
            for bi in range(b)
        ]
        for c in copies:
            c.start()
        for c in copies:
            c.wait()

    return sc_kernel(row_embed, col_embed)
